# baseline (speedup 1.0000x reference)
import functools

import jax
import jax.numpy as jnp
from jax import lax
from jax.experimental import pallas as pl
from jax.experimental.pallas import tpu as pltpu
from jax.experimental.pallas import tpu_sc as plsc

KK = 20
EPSV = 1e-5
NEG = -1e30
POS = 1e30

B, N = 4, 2048
BN = B * N
NE = BN * KK
NW = 32
EPW = NE // NW
ECH = 128
R_T = 256
R_K = 512
P_T = 128


def _dotf(a, b):
  return jax.lax.dot_general(
      a, b, (((1,), (0,)), ((), ())),
      preferred_element_type=jnp.float32)


def _topk_from_pd(xb, xf, b, idx_ref):
  g = jax.lax.dot_general(
      xb, xf, (((1,), (1,)), ((), ())),
      preferred_element_type=jnp.float32)
  xxb = jnp.sum(xb * xb, axis=1, keepdims=True)
  xxf = jnp.sum(xf * xf, axis=1, keepdims=True).T
  pd = -xxb - (-2.0 * g) - xxf
  colsf = lax.broadcasted_iota(jnp.int32, pd.shape, 1).astype(jnp.float32)
  nf = jnp.float32(N)
  base = b * N
  idx_acc = jnp.zeros((pd.shape[0], KK), jnp.int32)
  work = pd
  for t in range(KK):
    m = jnp.max(work, axis=1, keepdims=True)
    sel = jnp.where(work == m, colsf, nf)
    jf = jnp.min(sel, axis=1, keepdims=True)
    idx_acc = jnp.where(
        lax.broadcasted_iota(jnp.int32, idx_acc.shape, 1) == t,
        jf.astype(jnp.int32) + base, idx_acc)
    if t < KK - 1:
      work = jnp.where(colsf == jf, NEG, work)
  idx_ref[0] = idx_acc


def _knn_body(half, xblk_ref, xfull_ref, idx_ref):
  b = pl.program_id(0) + half * (B // 2)
  _topk_from_pd(xblk_ref[0], xfull_ref[0], b, idx_ref)


def _knn_call(xp, half):
  cp = xp.shape[2]
  b2 = B // 2
  return pl.pallas_call(
      functools.partial(_knn_body, half),
      grid=(b2, N // R_K),
      in_specs=[
          pl.BlockSpec((1, R_K, cp), lambda b, r: (b + half * b2, r, 0)),
          pl.BlockSpec((1, N, cp), lambda b, r: (b + half * b2, 0, 0)),
      ],
      out_specs=pl.BlockSpec((1, R_K, KK), lambda b, r: (b, r, 0)),
      out_shape=jax.ShapeDtypeStruct((b2, N, KK), jnp.int32),
  )(xp, xp)


SCH_E = 2 * ECH


def _gather_body(epw, xp_hbm, idx_hbm, nbr_hbm, idx_v, buf_a, buf_b,
                 sem_a, sem_b):
  wid = lax.axis_index("s") * 2 + lax.axis_index("c")
  base_e = wid * epw
  pltpu.sync_copy(idx_hbm.at[pl.ds(base_e, epw)], idx_v)
  nsc = epw // SCH_E

  def fire(sc, buf, sem):
    e0 = sc * SCH_E
    pltpu.async_copy(
        xp_hbm.at[idx_v.at[pl.ds(e0, ECH)]], buf.at[pl.ds(0, ECH)], sem)
    pltpu.async_copy(
        xp_hbm.at[idx_v.at[pl.ds(e0 + ECH, ECH)]],
        buf.at[pl.ds(ECH, ECH)], sem)

  def drain_out(sc, buf, sem):
    pltpu.make_async_copy(
        xp_hbm.at[idx_v.at[pl.ds(0, ECH)]], buf.at[pl.ds(0, ECH)],
        sem).wait()
    pltpu.make_async_copy(
        xp_hbm.at[idx_v.at[pl.ds(0, ECH)]], buf.at[pl.ds(ECH, ECH)],
        sem).wait()
    pltpu.sync_copy(buf, nbr_hbm.at[pl.ds(base_e + sc * SCH_E, SCH_E)])

  fire(0, buf_a, sem_a)

  def body(p, carry):
    sc = p * 2
    fire(sc + 1, buf_b, sem_b)
    drain_out(sc, buf_a, sem_a)

    @pl.when(sc + 2 < nsc)
    def _():
      fire(sc + 2, buf_a, sem_a)

    drain_out(sc + 1, buf_b, sem_b)
    return carry

  lax.fori_loop(0, nsc // 2, body, 0)


def _gather_call(xp_flat, idx_flat):
  cp = xp_flat.shape[1]
  ne = idx_flat.shape[0]
  epw = ne // NW
  mesh = plsc.VectorSubcoreMesh(core_axis_name="c", subcore_axis_name="s")
  kern = functools.partial(
      pl.kernel,
      mesh=mesh,
      out_type=jax.ShapeDtypeStruct((ne, cp), jnp.float32),
      scratch_types=[
          pltpu.VMEM((epw,), jnp.int32),
          pltpu.VMEM((SCH_E, cp), jnp.float32),
          pltpu.VMEM((SCH_E, cp), jnp.float32),
          pltpu.SemaphoreType.DMA,
          pltpu.SemaphoreType.DMA,
      ],
  )(functools.partial(_gather_body, epw))
  return kern(xp_flat, idx_flat)


def _conv_body(nbr_ref, xp_ref, wa_ref, wb_ref, hmax_ref, hmin_ref,
               stat_ref, acc_ref):
  r = pl.program_id(0)
  nr = pl.num_programs(0)

  @pl.when(r == 0)
  def _init():
    acc_ref[...] = jnp.zeros_like(acc_ref)

  cp = xp_ref.shape[1]
  nbr = nbr_ref[...]
  xc = xp_ref[...]
  xcb = jnp.broadcast_to(xc[:, None, :], (P_T, KK, cp)).reshape(P_T * KK, cp)
  fd = nbr - xcb
  h = _dotf(fd, wa_ref[...]) + _dotf(xcb, wb_ref[...])
  o = h.shape[1]
  h3 = h.reshape(P_T, KK, o)
  hmax_ref[...] = jnp.max(h3, axis=1)
  hmin_ref[...] = jnp.min(h3, axis=1)
  acc_ref[0] += jnp.sum(h, axis=0)
  acc_ref[1] += jnp.sum(h * h, axis=0)

  @pl.when(r == nr - 1)
  def _fin():
    stat_ref[...] = acc_ref[...]


def _conv_call(nbr, xp_flat, wat, wbt, off_blocks=0, n_points=BN):
  cp = xp_flat.shape[1]
  o = wat.shape[1]
  return pl.pallas_call(
      _conv_body,
      grid=(n_points // P_T,),
      in_specs=[
          pl.BlockSpec((P_T * KK, cp), lambda r: (r, 0)),
          pl.BlockSpec((P_T, cp), lambda r: (r + off_blocks, 0)),
          pl.BlockSpec((cp, o), lambda r: (0, 0)),
          pl.BlockSpec((cp, o), lambda r: (0, 0)),
      ],
      out_specs=[
          pl.BlockSpec((P_T, o), lambda r: (r, 0)),
          pl.BlockSpec((P_T, o), lambda r: (r, 0)),
          pl.BlockSpec((8, o), lambda r: (0, 0)),
      ],
      out_shape=[
          jax.ShapeDtypeStruct((n_points, o), jnp.float32),
          jax.ShapeDtypeStruct((n_points, o), jnp.float32),
          jax.ShapeDtypeStruct((8, o), jnp.float32),
      ],
      scratch_shapes=[pltpu.VMEM((8, o), jnp.float32)],
  )(nbr, xp_flat, wat, wbt)


def _bn_body(hmax_ref, hmin_ref, sa_ref, sb_ref, g_ref, b_ref, out_ref):
  o = hmax_ref.shape[1]
  opad = out_ref.shape[1]
  cnt = jnp.float32(BN * KK)
  s0 = sa_ref[0] + sb_ref[0]
  s1 = sa_ref[1] + sb_ref[1]
  m = s0 / cnt
  v = s1 / cnt - m * m
  gv = g_ref[...]
  ext = jnp.where((gv >= 0)[None, :], hmax_ref[...], hmin_ref[...])
  y = (ext - m[None, :]) / jnp.sqrt(v + EPSV)[None, :] * gv[None, :] + \
      b_ref[...][None, :]
  y = jnp.maximum(y, 0.0)
  if opad > o:
    out_ref[:, :o] = y
    out_ref[:, o:] = jnp.zeros((y.shape[0], opad - o), jnp.float32)
  else:
    out_ref[...] = y


def _bn_call(hmax, hmin, stat_a, stat_b, g, b):
  o = hmax.shape[1]
  opad = max(o, 128)
  rt = 1024
  return pl.pallas_call(
      _bn_body,
      grid=(BN // rt,),
      in_specs=[
          pl.BlockSpec((rt, o), lambda r: (r, 0)),
          pl.BlockSpec((rt, o), lambda r: (r, 0)),
          pl.BlockSpec((8, o), lambda r: (0, 0)),
          pl.BlockSpec((8, o), lambda r: (0, 0)),
          pl.BlockSpec((o,), lambda r: (0,)),
          pl.BlockSpec((o,), lambda r: (0,)),
      ],
      out_specs=pl.BlockSpec((rt, opad), lambda r: (r, 0)),
      out_shape=jax.ShapeDtypeStruct((BN, opad), jnp.float32),
  )(hmax, hmin, stat_a, stat_b, g, b)


def _final_body(x1_ref, x2_ref, x3_ref, x4_ref, w1_ref, w2_ref, w3_ref,
                w4_ref, g_ref, b_ref, out_ref, acc_ref):
  bb = pl.program_id(0)
  r = pl.program_id(1)
  nr = pl.num_programs(1)

  @pl.when(jnp.logical_and(bb == 0, r == 0))
  def _init():
    acc_ref[0:2] = jnp.zeros((2, 1024), jnp.float32)
    acc_ref[2:2 + B] = jnp.full((B, 1024), NEG, jnp.float32)
    acc_ref[2 + B:2 + 2 * B] = jnp.full((B, 1024), POS, jnp.float32)

  h = (_dotf(x1_ref[0], w1_ref[...]) + _dotf(x2_ref[0], w2_ref[...]) +
       _dotf(x3_ref[0], w3_ref[...]) + _dotf(x4_ref[0], w4_ref[...]))
  acc_ref[0] += jnp.sum(h, axis=0)
  acc_ref[1] += jnp.sum(h * h, axis=0)
  bmax = jnp.max(h, axis=0, keepdims=True)
  bmin = jnp.min(h, axis=0, keepdims=True)
  acc_ref[pl.ds(2 + bb, 1)] = jnp.maximum(acc_ref[pl.ds(2 + bb, 1)], bmax)
  acc_ref[pl.ds(2 + B + bb, 1)] = jnp.minimum(
      acc_ref[pl.ds(2 + B + bb, 1)], bmin)

  @pl.when(jnp.logical_and(bb == B - 1, r == nr - 1))
  def _fin():
    cnt = jnp.float32(BN)
    m = acc_ref[0] / cnt
    v = acc_ref[1] / cnt - m * m
    gv = g_ref[...]
    ext = jnp.where((gv >= 0)[None, :], acc_ref[2:2 + B],
                    acc_ref[2 + B:2 + 2 * B])
    y = (ext - m[None, :]) / jnp.sqrt(v + EPSV)[None, :] * gv[None, :] + \
        b_ref[...][None, :]
    out_ref[...] = jnp.maximum(y, 0.0)


def _final_call(x1, x2, x3, x4, w5, g5, b5):
  w1t = jnp.pad(w5[:, :64], ((0, 0), (0, 64))).T
  w2t = jnp.pad(w5[:, 64:128], ((0, 0), (0, 64))).T
  w3t = w5[:, 128:256].T
  w4t = w5[:, 256:].T
  return pl.pallas_call(
      _final_body,
      grid=(B, N // R_T),
      in_specs=[
          pl.BlockSpec((1, R_T, 128), lambda b, r: (b, r, 0)),
          pl.BlockSpec((1, R_T, 128), lambda b, r: (b, r, 0)),
          pl.BlockSpec((1, R_T, 128), lambda b, r: (b, r, 0)),
          pl.BlockSpec((1, R_T, 256), lambda b, r: (b, r, 0)),
          pl.BlockSpec((128, 1024), lambda b, r: (0, 0)),
          pl.BlockSpec((128, 1024), lambda b, r: (0, 0)),
          pl.BlockSpec((128, 1024), lambda b, r: (0, 0)),
          pl.BlockSpec((256, 1024), lambda b, r: (0, 0)),
          pl.BlockSpec((1024,), lambda b, r: (0,)),
          pl.BlockSpec((1024,), lambda b, r: (0,)),
      ],
      out_specs=pl.BlockSpec((B, 1024), lambda b, r: (0, 0)),
      out_shape=jax.ShapeDtypeStruct((B, 1024), jnp.float32),
      scratch_shapes=[pltpu.VMEM((2 + 2 * B, 1024), jnp.float32)],
  )(x1, x2, x3, x4, w1t, w2t, w3t, w4t, g5, b5)


def _wsplit(w, cpad):
  cw = w.shape[1] // 2
  wat = jnp.pad(w[:, :cw], ((0, 0), (0, cpad - cw))).T
  wbt = jnp.pad(w[:, cw:], ((0, 0), (0, cpad - cw))).T
  return wat, wbt


def _knn_gather_conv(xp, w):
  cpad = xp.shape[2]
  wat, wbt = _wsplit(w, cpad)
  xp_flat = xp.reshape(BN, cpad)
  ne2 = NE // 2
  np2 = BN // 2
  idx_a = _knn_call(xp, 0)
  nbr_a = _gather_call(xp_flat, idx_a.reshape(ne2))
  idx_b = _knn_call(xp, 1)
  nbr_b = _gather_call(xp_flat, idx_b.reshape(ne2))
  hx_a, hn_a, st_a = _conv_call(nbr_a, xp_flat, wat, wbt, 0, np2)
  hx_b, hn_b, st_b = _conv_call(
      nbr_b, xp_flat, wat, wbt, np2 // P_T, np2)
  hx = jnp.concatenate([hx_a, hx_b], axis=0)
  hn = jnp.concatenate([hn_a, hn_b], axis=0)
  return hx, hn, st_a, st_b


def kernel(x, W1, W2, W3, W4, W5, g1, b1, g2, b2, g3, b3, g4, b4, g5, b5):
  xt = jnp.transpose(x, (0, 2, 1))
  xp = jnp.pad(xt, ((0, 0), (0, 0), (0, 125)))
  hx1, hn1, sa1, sb1 = _knn_gather_conv(xp, W1)
  x1 = _bn_call(hx1, hn1, sa1, sb1, g1, b1).reshape(B, N, 128)
  hx2, hn2, sa2, sb2 = _knn_gather_conv(x1, W2)
  x2 = _bn_call(hx2, hn2, sa2, sb2, g2, b2).reshape(B, N, 128)
  hx3, hn3, sa3, sb3 = _knn_gather_conv(x2, W3)
  x3 = _bn_call(hx3, hn3, sa3, sb3, g3, b3).reshape(B, N, 128)
  hx4, hn4, sa4, sb4 = _knn_gather_conv(x3, W4)
  x4 = _bn_call(hx4, hn4, sa4, sb4, g4, b4).reshape(B, N, 256)
  return _final_call(x1, x2, x3, x4, W5, g5, b5)


# speedup vs baseline: 1.0240x; 1.0240x over previous
import functools

import jax
import jax.numpy as jnp
from jax import lax
from jax.experimental import pallas as pl
from jax.experimental.pallas import tpu as pltpu
from jax.experimental.pallas import tpu_sc as plsc

KK = 20
EPSV = 1e-5
NEG = -1e30
POS = 1e30

B, N = 4, 2048
BN = B * N
NE = BN * KK
NW = 32
EPW = NE // NW
ECH = 128
R_T = 256
R_K = 256
P_T = 128


def _dotf(a, b):
  return jax.lax.dot_general(
      a, b, (((1,), (0,)), ((), ())),
      preferred_element_type=jnp.float32)


def _topk_from_pd(xb, xf, b, idx_ref):
  g = jax.lax.dot_general(
      xb, xf, (((1,), (1,)), ((), ())),
      preferred_element_type=jnp.float32)
  xxb = jnp.sum(xb * xb, axis=1, keepdims=True)
  xxf = jnp.sum(xf * xf, axis=1, keepdims=True).T
  pd = -xxb - (-2.0 * g) - xxf
  colsf = lax.broadcasted_iota(jnp.int32, pd.shape, 1).astype(jnp.float32)
  nf = jnp.float32(N)
  base = b * N
  idx_acc = jnp.zeros((pd.shape[0], KK), jnp.int32)
  work = pd
  for t in range(KK):
    m = jnp.max(work, axis=1, keepdims=True)
    sel = jnp.where(work == m, colsf, nf)
    jf = jnp.min(sel, axis=1, keepdims=True)
    idx_acc = jnp.where(
        lax.broadcasted_iota(jnp.int32, idx_acc.shape, 1) == t,
        jf.astype(jnp.int32) + base, idx_acc)
    if t < KK - 1:
      work = jnp.where(colsf == jf, NEG, work)
  idx_ref[0] = idx_acc


def _knn_body(half, xblk_ref, xfull_ref, idx_ref):
  b = pl.program_id(0) + half * (B // 2)
  _topk_from_pd(xblk_ref[0], xfull_ref[0], b, idx_ref)


def _knn_call(xp, half):
  cp = xp.shape[2]
  b2 = B // 2
  return pl.pallas_call(
      functools.partial(_knn_body, half),
      grid=(b2, N // R_K),
      in_specs=[
          pl.BlockSpec((1, R_K, cp), lambda b, r: (b + half * b2, r, 0)),
          pl.BlockSpec((1, N, cp), lambda b, r: (b + half * b2, 0, 0)),
      ],
      out_specs=pl.BlockSpec((1, R_K, KK), lambda b, r: (b, r, 0)),
      out_shape=jax.ShapeDtypeStruct((b2, N, KK), jnp.int32),
  )(xp, xp)


SCH_E = 2 * ECH


def _gather_body(epw, xp_hbm, idx_hbm, nbr_hbm, idx_v, buf_a, buf_b,
                 sem_a, sem_b):
  wid = lax.axis_index("s") * 2 + lax.axis_index("c")
  base_e = wid * epw
  pltpu.sync_copy(idx_hbm.at[pl.ds(base_e, epw)], idx_v)
  nsc = epw // SCH_E

  def fire(sc, buf, sem):
    e0 = sc * SCH_E
    pltpu.async_copy(
        xp_hbm.at[idx_v.at[pl.ds(e0, ECH)]], buf.at[pl.ds(0, ECH)], sem)
    pltpu.async_copy(
        xp_hbm.at[idx_v.at[pl.ds(e0 + ECH, ECH)]],
        buf.at[pl.ds(ECH, ECH)], sem)

  def drain_out(sc, buf, sem):
    pltpu.make_async_copy(
        xp_hbm.at[idx_v.at[pl.ds(0, ECH)]], buf.at[pl.ds(0, ECH)],
        sem).wait()
    pltpu.make_async_copy(
        xp_hbm.at[idx_v.at[pl.ds(0, ECH)]], buf.at[pl.ds(ECH, ECH)],
        sem).wait()
    pltpu.sync_copy(buf, nbr_hbm.at[pl.ds(base_e + sc * SCH_E, SCH_E)])

  fire(0, buf_a, sem_a)

  def body(p, carry):
    sc = p * 2
    fire(sc + 1, buf_b, sem_b)
    drain_out(sc, buf_a, sem_a)

    @pl.when(sc + 2 < nsc)
    def _():
      fire(sc + 2, buf_a, sem_a)

    drain_out(sc + 1, buf_b, sem_b)
    return carry

  lax.fori_loop(0, nsc // 2, body, 0)


def _gather_call(xp_flat, idx_flat):
  cp = xp_flat.shape[1]
  ne = idx_flat.shape[0]
  epw = ne // NW
  mesh = plsc.VectorSubcoreMesh(core_axis_name="c", subcore_axis_name="s")
  kern = functools.partial(
      pl.kernel,
      mesh=mesh,
      out_type=jax.ShapeDtypeStruct((ne, cp), jnp.float32),
      scratch_types=[
          pltpu.VMEM((epw,), jnp.int32),
          pltpu.VMEM((SCH_E, cp), jnp.float32),
          pltpu.VMEM((SCH_E, cp), jnp.float32),
          pltpu.SemaphoreType.DMA,
          pltpu.SemaphoreType.DMA,
      ],
  )(functools.partial(_gather_body, epw))
  return kern(xp_flat, idx_flat)


def _conv_body(nbr_ref, xp_ref, wa_ref, wb_ref, hmax_ref, hmin_ref,
               stat_ref, acc_ref):
  r = pl.program_id(0)
  nr = pl.num_programs(0)

  @pl.when(r == 0)
  def _init():
    acc_ref[...] = jnp.zeros_like(acc_ref)

  cp = xp_ref.shape[1]
  nbr = nbr_ref[...]
  xc = xp_ref[...]
  xcb = jnp.broadcast_to(xc[:, None, :], (P_T, KK, cp)).reshape(P_T * KK, cp)
  fd = nbr - xcb
  h = _dotf(fd, wa_ref[...]) + _dotf(xcb, wb_ref[...])
  o = h.shape[1]
  h3 = h.reshape(P_T, KK, o)
  hmax_ref[...] = jnp.max(h3, axis=1)
  hmin_ref[...] = jnp.min(h3, axis=1)
  acc_ref[0] += jnp.sum(h, axis=0)
  acc_ref[1] += jnp.sum(h * h, axis=0)

  @pl.when(r == nr - 1)
  def _fin():
    stat_ref[...] = acc_ref[...]


def _conv_call(nbr, xp_flat, wat, wbt, off_blocks=0, n_points=BN):
  cp = xp_flat.shape[1]
  o = wat.shape[1]
  return pl.pallas_call(
      _conv_body,
      grid=(n_points // P_T,),
      in_specs=[
          pl.BlockSpec((P_T * KK, cp), lambda r: (r, 0)),
          pl.BlockSpec((P_T, cp), lambda r: (r + off_blocks, 0)),
          pl.BlockSpec((cp, o), lambda r: (0, 0)),
          pl.BlockSpec((cp, o), lambda r: (0, 0)),
      ],
      out_specs=[
          pl.BlockSpec((P_T, o), lambda r: (r, 0)),
          pl.BlockSpec((P_T, o), lambda r: (r, 0)),
          pl.BlockSpec((8, o), lambda r: (0, 0)),
      ],
      out_shape=[
          jax.ShapeDtypeStruct((n_points, o), jnp.float32),
          jax.ShapeDtypeStruct((n_points, o), jnp.float32),
          jax.ShapeDtypeStruct((8, o), jnp.float32),
      ],
      scratch_shapes=[pltpu.VMEM((8, o), jnp.float32)],
  )(nbr, xp_flat, wat, wbt)


def _bn_body(hmax_ref, hmin_ref, sa_ref, sb_ref, g_ref, b_ref, out_ref):
  o = hmax_ref.shape[1]
  opad = out_ref.shape[1]
  cnt = jnp.float32(BN * KK)
  s0 = sa_ref[0] + sb_ref[0]
  s1 = sa_ref[1] + sb_ref[1]
  m = s0 / cnt
  v = s1 / cnt - m * m
  gv = g_ref[...]
  ext = jnp.where((gv >= 0)[None, :], hmax_ref[...], hmin_ref[...])
  y = (ext - m[None, :]) / jnp.sqrt(v + EPSV)[None, :] * gv[None, :] + \
      b_ref[...][None, :]
  y = jnp.maximum(y, 0.0)
  if opad > o:
    out_ref[:, :o] = y
    out_ref[:, o:] = jnp.zeros((y.shape[0], opad - o), jnp.float32)
  else:
    out_ref[...] = y


def _bn_call(hmax, hmin, stat_a, stat_b, g, b):
  o = hmax.shape[1]
  opad = max(o, 128)
  rt = 1024
  return pl.pallas_call(
      _bn_body,
      grid=(BN // rt,),
      in_specs=[
          pl.BlockSpec((rt, o), lambda r: (r, 0)),
          pl.BlockSpec((rt, o), lambda r: (r, 0)),
          pl.BlockSpec((8, o), lambda r: (0, 0)),
          pl.BlockSpec((8, o), lambda r: (0, 0)),
          pl.BlockSpec((o,), lambda r: (0,)),
          pl.BlockSpec((o,), lambda r: (0,)),
      ],
      out_specs=pl.BlockSpec((rt, opad), lambda r: (r, 0)),
      out_shape=jax.ShapeDtypeStruct((BN, opad), jnp.float32),
  )(hmax, hmin, stat_a, stat_b, g, b)


def _final_body(x1_ref, x2_ref, x3_ref, x4_ref, w1_ref, w2_ref, w3_ref,
                w4_ref, g_ref, b_ref, out_ref, acc_ref):
  bb = pl.program_id(0)
  r = pl.program_id(1)
  nr = pl.num_programs(1)

  @pl.when(jnp.logical_and(bb == 0, r == 0))
  def _init():
    acc_ref[0:2] = jnp.zeros((2, 1024), jnp.float32)
    acc_ref[2:2 + B] = jnp.full((B, 1024), NEG, jnp.float32)
    acc_ref[2 + B:2 + 2 * B] = jnp.full((B, 1024), POS, jnp.float32)

  h = (_dotf(x1_ref[0], w1_ref[...]) + _dotf(x2_ref[0], w2_ref[...]) +
       _dotf(x3_ref[0], w3_ref[...]) + _dotf(x4_ref[0], w4_ref[...]))
  acc_ref[0] += jnp.sum(h, axis=0)
  acc_ref[1] += jnp.sum(h * h, axis=0)
  bmax = jnp.max(h, axis=0, keepdims=True)
  bmin = jnp.min(h, axis=0, keepdims=True)
  acc_ref[pl.ds(2 + bb, 1)] = jnp.maximum(acc_ref[pl.ds(2 + bb, 1)], bmax)
  acc_ref[pl.ds(2 + B + bb, 1)] = jnp.minimum(
      acc_ref[pl.ds(2 + B + bb, 1)], bmin)

  @pl.when(jnp.logical_and(bb == B - 1, r == nr - 1))
  def _fin():
    cnt = jnp.float32(BN)
    m = acc_ref[0] / cnt
    v = acc_ref[1] / cnt - m * m
    gv = g_ref[...]
    ext = jnp.where((gv >= 0)[None, :], acc_ref[2:2 + B],
                    acc_ref[2 + B:2 + 2 * B])
    y = (ext - m[None, :]) / jnp.sqrt(v + EPSV)[None, :] * gv[None, :] + \
        b_ref[...][None, :]
    out_ref[...] = jnp.maximum(y, 0.0)


def _final_call(x1, x2, x3, x4, w5, g5, b5):
  w1t = jnp.pad(w5[:, :64], ((0, 0), (0, 64))).T
  w2t = jnp.pad(w5[:, 64:128], ((0, 0), (0, 64))).T
  w3t = w5[:, 128:256].T
  w4t = w5[:, 256:].T
  return pl.pallas_call(
      _final_body,
      grid=(B, N // R_T),
      in_specs=[
          pl.BlockSpec((1, R_T, 128), lambda b, r: (b, r, 0)),
          pl.BlockSpec((1, R_T, 128), lambda b, r: (b, r, 0)),
          pl.BlockSpec((1, R_T, 128), lambda b, r: (b, r, 0)),
          pl.BlockSpec((1, R_T, 256), lambda b, r: (b, r, 0)),
          pl.BlockSpec((128, 1024), lambda b, r: (0, 0)),
          pl.BlockSpec((128, 1024), lambda b, r: (0, 0)),
          pl.BlockSpec((128, 1024), lambda b, r: (0, 0)),
          pl.BlockSpec((256, 1024), lambda b, r: (0, 0)),
          pl.BlockSpec((1024,), lambda b, r: (0,)),
          pl.BlockSpec((1024,), lambda b, r: (0,)),
      ],
      out_specs=pl.BlockSpec((B, 1024), lambda b, r: (0, 0)),
      out_shape=jax.ShapeDtypeStruct((B, 1024), jnp.float32),
      scratch_shapes=[pltpu.VMEM((2 + 2 * B, 1024), jnp.float32)],
  )(x1, x2, x3, x4, w1t, w2t, w3t, w4t, g5, b5)


def _wsplit(w, cpad):
  cw = w.shape[1] // 2
  wat = jnp.pad(w[:, :cw], ((0, 0), (0, cpad - cw))).T
  wbt = jnp.pad(w[:, cw:], ((0, 0), (0, cpad - cw))).T
  return wat, wbt


def _knn_gather_conv(xp, w):
  cpad = xp.shape[2]
  wat, wbt = _wsplit(w, cpad)
  xp_flat = xp.reshape(BN, cpad)
  ne2 = NE // 2
  np2 = BN // 2
  idx_a = _knn_call(xp, 0)
  nbr_a = _gather_call(xp_flat, idx_a.reshape(ne2))
  idx_b = _knn_call(xp, 1)
  nbr_b = _gather_call(xp_flat, idx_b.reshape(ne2))
  hx_a, hn_a, st_a = _conv_call(nbr_a, xp_flat, wat, wbt, 0, np2)
  hx_b, hn_b, st_b = _conv_call(
      nbr_b, xp_flat, wat, wbt, np2 // P_T, np2)
  hx = jnp.concatenate([hx_a, hx_b], axis=0)
  hn = jnp.concatenate([hn_a, hn_b], axis=0)
  return hx, hn, st_a, st_b


def kernel(x, W1, W2, W3, W4, W5, g1, b1, g2, b2, g3, b3, g4, b4, g5, b5):
  xt = jnp.transpose(x, (0, 2, 1))
  xp = jnp.pad(xt, ((0, 0), (0, 0), (0, 125)))
  hx1, hn1, sa1, sb1 = _knn_gather_conv(xp, W1)
  x1 = _bn_call(hx1, hn1, sa1, sb1, g1, b1).reshape(B, N, 128)
  hx2, hn2, sa2, sb2 = _knn_gather_conv(x1, W2)
  x2 = _bn_call(hx2, hn2, sa2, sb2, g2, b2).reshape(B, N, 128)
  hx3, hn3, sa3, sb3 = _knn_gather_conv(x2, W3)
  x3 = _bn_call(hx3, hn3, sa3, sb3, g3, b3).reshape(B, N, 128)
  hx4, hn4, sa4, sb4 = _knn_gather_conv(x3, W4)
  x4 = _bn_call(hx4, hn4, sa4, sb4, g4, b4).reshape(B, N, 256)
  return _final_call(x1, x2, x3, x4, W5, g5, b5)
